# Initial kernel scaffold; baseline (speedup 1.0000x reference)
import jax
import jax.numpy as jnp
from jax.experimental import pallas as pl


def kernel(x, visible_ids, W, b, mask_tokens, pos_embeds, view_embeds):
    raise NotImplementedError("write your pallas kernel here")


# speedup vs baseline: 1.3698x; 1.3698x over previous
import functools

import jax
import jax.numpy as jnp
from jax import lax
from jax.experimental import pallas as pl
from jax.experimental.pallas import tpu as pltpu
from jax.experimental.pallas import tpu_sc as plsc


def _tc_body(np2, x_ref, wt_ref, b_ref, ids_ref, m_ref, p_ref, vw_ref,
             v_ref, gid_ref, out_ref):
    bidx = pl.program_id(0)
    ids2 = ids_ref[0]
    pe = p_ref[...] + vw_ref[...]
    out_ref[0] = m_ref[...] + pe
    ohT = (ids2 == lax.broadcasted_iota(jnp.int32, (np2, ids2.shape[1]), 0)
           ).astype(jnp.float32)
    pe_g = lax.dot_general(ohT, pe, (((0,), (0,)), ((), ())),
                           precision=lax.Precision.HIGHEST,
                           preferred_element_type=jnp.float32)
    xd = jnp.dot(x_ref[0], wt_ref[...],
                 precision=lax.Precision.HIGHEST,
                 preferred_element_type=jnp.float32)
    v_ref[0] = xd + pe_g + b_ref[...]
    gid_ref[0] = ids2 + bidx * np2


def _sc_scatter_body(rpw, ch, v_hbm, gid_hbm, out_hbm,
                     idx0, buf0, idx1, buf1, sem0, sem1):
    c = lax.axis_index("c")
    s = lax.axis_index("s")
    wid = s * 2 + c
    base = wid * rpw
    bufs = ((idx0, buf0, sem0), (idx1, buf1, sem1))
    for k in range(rpw // ch):
        idx, buf, sem = bufs[k % 2]
        off = base + k * ch
        pltpu.sync_copy(gid_hbm.at[pl.ds(off, ch)], idx)
        pltpu.sync_copy(v_hbm.at[pl.ds(off, ch)], buf)
        pltpu.async_copy(buf, out_hbm.at[idx], sem).wait()


def kernel(x, visible_ids, W, b, mask_tokens, pos_embeds, view_embeds):
    B, NV, ED = x.shape
    DD = W.shape[0]
    NP2 = mask_tokens.shape[1]

    wt = W.T
    b2 = b.reshape(1, DD)
    ids3 = visible_ids.reshape(B, 1, NV)
    m2 = mask_tokens.reshape(NP2, DD)
    p2 = pos_embeds.reshape(NP2, DD)
    vw2 = view_embeds.reshape(NP2, DD)

    v, gid, out_init = pl.pallas_call(
        functools.partial(_tc_body, NP2),
        grid=(B,),
        in_specs=[
            pl.BlockSpec((1, NV, ED), lambda i: (i, 0, 0)),
            pl.BlockSpec((ED, DD), lambda i: (0, 0)),
            pl.BlockSpec((1, DD), lambda i: (0, 0)),
            pl.BlockSpec((1, 1, NV), lambda i: (i, 0, 0)),
            pl.BlockSpec((NP2, DD), lambda i: (0, 0)),
            pl.BlockSpec((NP2, DD), lambda i: (0, 0)),
            pl.BlockSpec((NP2, DD), lambda i: (0, 0)),
        ],
        out_specs=[
            pl.BlockSpec((1, NV, DD), lambda i: (i, 0, 0)),
            pl.BlockSpec((1, 1, NV), lambda i: (i, 0, 0)),
            pl.BlockSpec((1, NP2, DD), lambda i: (i, 0, 0)),
        ],
        out_shape=[
            jax.ShapeDtypeStruct((B, NV, DD), jnp.float32),
            jax.ShapeDtypeStruct((B, 1, NV), jnp.int32),
            jax.ShapeDtypeStruct((B, NP2, DD), jnp.float32),
        ],
        compiler_params=pltpu.CompilerParams(
            dimension_semantics=("arbitrary",)),
    )(x, wt, b2, ids3, m2, p2, vw2)

    info = plsc.get_sparse_core_info()
    nw = info.num_cores * info.num_subcores
    rows = B * NV
    rpw = rows // nw
    ch = 96

    mesh = plsc.VectorSubcoreMesh(core_axis_name="c", subcore_axis_name="s")
    sc_scatter = functools.partial(
        pl.kernel,
        out_type=(),
        mesh=mesh,
        scratch_types=[
            pltpu.VMEM((ch,), jnp.int32),
            pltpu.VMEM((ch, DD), jnp.float32),
            pltpu.VMEM((ch,), jnp.int32),
            pltpu.VMEM((ch, DD), jnp.float32),
            pltpu.SemaphoreType.DMA,
            pltpu.SemaphoreType.DMA,
        ],
    )(functools.partial(_sc_scatter_body, rpw, ch))

    out_ref = jax.new_ref(out_init.reshape(B * NP2, DD))
    sc_scatter(v.reshape(rows, DD), gid.reshape(rows), out_ref)
    return jax.freeze(out_ref).reshape(B, NP2, DD)


# speedup vs baseline: 2.6835x; 1.9591x over previous
import functools

import jax
import jax.numpy as jnp
from jax import lax
from jax.experimental import pallas as pl
from jax.experimental.pallas import tpu as pltpu
from jax.experimental.pallas import tpu_sc as plsc


def _tc_body(np2, x_ref, wt_ref, b_ref, ids_ref, idc_ref, m_ref, p_ref,
             vw_ref, v_ref, gid_ref, out_ref):
    bidx = pl.program_id(0)
    nv = ids_ref.shape[2]
    pe = p_ref[...] + vw_ref[...]
    out_ref[0] = m_ref[...] + pe
    oh = (idc_ref[0] == lax.broadcasted_iota(jnp.int32, (nv, np2), 1)
          ).astype(jnp.float32)
    pe_g = jnp.dot(oh, pe,
                   preferred_element_type=jnp.float32)
    xd = jnp.dot(x_ref[0], wt_ref[...],
                 preferred_element_type=jnp.float32)
    v_ref[0] = xd + pe_g + b_ref[...]
    gid_ref[0] = ids_ref[0] + bidx * np2


def _sc_scatter_body(rpw, ch, v_hbm, gid_hbm, out_hbm,
                     idx0, buf0, idx1, buf1, sem0, sem1):
    c = lax.axis_index("c")
    s = lax.axis_index("s")
    wid = s * 2 + c
    base = wid * rpw
    bufs = ((idx0, buf0, sem0), (idx1, buf1, sem1))
    for k in range(rpw // ch):
        idx, buf, sem = bufs[k % 2]
        off = base + k * ch
        pltpu.sync_copy(gid_hbm.at[pl.ds(off, ch)], idx)
        pltpu.sync_copy(v_hbm.at[pl.ds(off, ch)], buf)
        pltpu.async_copy(buf, out_hbm.at[idx], sem).wait()


def kernel(x, visible_ids, W, b, mask_tokens, pos_embeds, view_embeds):
    B, NV, ED = x.shape
    DD = W.shape[0]
    NP2 = mask_tokens.shape[1]

    wt = W.T
    b2 = b.reshape(1, DD)
    ids3 = visible_ids.reshape(B, 1, NV)
    idc3 = visible_ids.reshape(B, NV, 1)
    m2 = mask_tokens.reshape(NP2, DD)
    p2 = pos_embeds.reshape(NP2, DD)
    vw2 = view_embeds.reshape(NP2, DD)

    v, gid, out_init = pl.pallas_call(
        functools.partial(_tc_body, NP2),
        grid=(B,),
        in_specs=[
            pl.BlockSpec((1, NV, ED), lambda i: (i, 0, 0)),
            pl.BlockSpec((ED, DD), lambda i: (0, 0)),
            pl.BlockSpec((1, DD), lambda i: (0, 0)),
            pl.BlockSpec((1, 1, NV), lambda i: (i, 0, 0)),
            pl.BlockSpec((1, NV, 1), lambda i: (i, 0, 0)),
            pl.BlockSpec((NP2, DD), lambda i: (0, 0)),
            pl.BlockSpec((NP2, DD), lambda i: (0, 0)),
            pl.BlockSpec((NP2, DD), lambda i: (0, 0)),
        ],
        out_specs=[
            pl.BlockSpec((1, NV, DD), lambda i: (i, 0, 0)),
            pl.BlockSpec((1, 1, NV), lambda i: (i, 0, 0)),
            pl.BlockSpec((1, NP2, DD), lambda i: (i, 0, 0)),
        ],
        out_shape=[
            jax.ShapeDtypeStruct((B, NV, DD), jnp.float32),
            jax.ShapeDtypeStruct((B, 1, NV), jnp.int32),
            jax.ShapeDtypeStruct((B, NP2, DD), jnp.float32),
        ],
        compiler_params=pltpu.CompilerParams(
            dimension_semantics=("arbitrary",)),
    )(x, wt, b2, ids3, idc3, m2, p2, vw2)

    info = plsc.get_sparse_core_info()
    nw = info.num_cores * info.num_subcores
    rows = B * NV
    rpw = rows // nw
    ch = 96

    mesh = plsc.VectorSubcoreMesh(core_axis_name="c", subcore_axis_name="s")
    sc_scatter = functools.partial(
        pl.kernel,
        out_type=(),
        mesh=mesh,
        scratch_types=[
            pltpu.VMEM((ch,), jnp.int32),
            pltpu.VMEM((ch, DD), jnp.float32),
            pltpu.VMEM((ch,), jnp.int32),
            pltpu.VMEM((ch, DD), jnp.float32),
            pltpu.SemaphoreType.DMA,
            pltpu.SemaphoreType.DMA,
        ],
    )(functools.partial(_sc_scatter_body, rpw, ch))

    out_ref = jax.new_ref(out_init.reshape(B * NP2, DD))
    sc_scatter(v.reshape(rows, DD), gid.reshape(rows), out_ref)
    return jax.freeze(out_ref).reshape(B, NP2, DD)


# speedup vs baseline: 2.8059x; 1.0456x over previous
import functools

import jax
import jax.numpy as jnp
from jax import lax
from jax.experimental import pallas as pl
from jax.experimental.pallas import tpu as pltpu
from jax.experimental.pallas import tpu_sc as plsc


def _base_body(m_ref, p_ref, vw_ref, base_ref):
    base_ref[...] = m_ref[...] + p_ref[...] + vw_ref[...]


def _tc_body(np2, x_ref, wt_ref, b_ref, ids_ref, idc_ref, p_ref, vw_ref,
             v_ref, gid_ref):
    bidx = pl.program_id(0)
    nv = ids_ref.shape[2]
    pe = p_ref[...] + vw_ref[...]
    oh = (idc_ref[0] == lax.broadcasted_iota(jnp.int32, (nv, np2), 1)
          ).astype(jnp.float32)
    pe_g = jnp.dot(oh, pe,
                   preferred_element_type=jnp.float32)
    xd = jnp.dot(x_ref[0], wt_ref[...],
                 preferred_element_type=jnp.float32)
    v_ref[0] = xd + pe_g + b_ref[...]
    gid_ref[0] = ids_ref[0] + bidx * np2


def _sc_fill_body(np2, base_hbm, out_hbm, base_sh, sem):
    c = lax.axis_index("c")
    s = lax.axis_index("s")
    wid = s * 2 + c
    @pl.when(s == 0)
    def _():
        pltpu.async_copy(base_hbm, base_sh, sem).wait()
    plsc.subcore_barrier()
    pltpu.sync_copy(base_sh, out_hbm.at[pl.ds((2 * wid) * np2, np2)])
    pltpu.sync_copy(base_sh, out_hbm.at[pl.ds((2 * wid + 1) * np2, np2)])


def _sc_scatter_body(rpw, ch, v_hbm, gid2_hbm, out_hbm,
                     idx2, buf0, buf1, semi0, semi1, semo0, semo1):
    c = lax.axis_index("c")
    s = lax.axis_index("s")
    wid = s * 2 + c
    base = wid * rpw
    nch = rpw // ch
    bufs = (buf0, buf1)
    semis = (semi0, semi1)
    semos = (semo0, semo1)
    pltpu.sync_copy(gid2_hbm.at[wid], idx2)
    in_cp = [None] * nch
    out_cp = [None] * nch
    in_cp[0] = pltpu.async_copy(v_hbm.at[pl.ds(base, ch)], buf0, semi0)
    for k in range(nch):
        if k + 1 < nch:
            if k - 1 >= 0:
                out_cp[k - 1].wait()
            in_cp[k + 1] = pltpu.async_copy(
                v_hbm.at[pl.ds(base + (k + 1) * ch, ch)],
                bufs[(k + 1) % 2], semis[(k + 1) % 2])
        in_cp[k].wait()
        out_cp[k] = pltpu.async_copy(bufs[k % 2], out_hbm.at[idx2.at[k]],
                                     semos[k % 2])
    out_cp[nch - 2].wait()
    out_cp[nch - 1].wait()


def kernel(x, visible_ids, W, b, mask_tokens, pos_embeds, view_embeds):
    B, NV, ED = x.shape
    DD = W.shape[0]
    NP2 = mask_tokens.shape[1]

    wt = W.T
    b2 = b.reshape(1, DD)
    ids3 = visible_ids.reshape(B, 1, NV)
    idc3 = visible_ids.reshape(B, NV, 1)
    m2 = mask_tokens.reshape(NP2, DD)
    p2 = pos_embeds.reshape(NP2, DD)
    vw2 = view_embeds.reshape(NP2, DD)

    base = pl.pallas_call(
        _base_body,
        out_shape=jax.ShapeDtypeStruct((NP2, DD), jnp.float32),
    )(m2, p2, vw2)

    v, gid = pl.pallas_call(
        functools.partial(_tc_body, NP2),
        grid=(B,),
        in_specs=[
            pl.BlockSpec((1, NV, ED), lambda i: (i, 0, 0)),
            pl.BlockSpec((ED, DD), lambda i: (0, 0)),
            pl.BlockSpec((1, DD), lambda i: (0, 0)),
            pl.BlockSpec((1, 1, NV), lambda i: (i, 0, 0)),
            pl.BlockSpec((1, NV, 1), lambda i: (i, 0, 0)),
            pl.BlockSpec((NP2, DD), lambda i: (0, 0)),
            pl.BlockSpec((NP2, DD), lambda i: (0, 0)),
        ],
        out_specs=[
            pl.BlockSpec((1, NV, DD), lambda i: (i, 0, 0)),
            pl.BlockSpec((1, 1, NV), lambda i: (i, 0, 0)),
        ],
        out_shape=[
            jax.ShapeDtypeStruct((B, NV, DD), jnp.float32),
            jax.ShapeDtypeStruct((B, 1, NV), jnp.int32),
        ],
        compiler_params=pltpu.CompilerParams(
            dimension_semantics=("arbitrary",)),
    )(x, wt, b2, ids3, idc3, p2, vw2)

    info = plsc.get_sparse_core_info()
    nw = info.num_cores * info.num_subcores
    rows = B * NV
    rpw = rows // nw
    ch = 96
    nch = rpw // ch

    mesh = plsc.VectorSubcoreMesh(core_axis_name="c", subcore_axis_name="s")

    sc_fill = functools.partial(
        pl.kernel,
        out_type=jax.ShapeDtypeStruct((B * NP2, DD), jnp.float32),
        mesh=mesh,
        scratch_types=[
            pltpu.VMEM_SHARED((NP2, DD), jnp.float32),
            pltpu.SemaphoreType.DMA,
        ],
    )(functools.partial(_sc_fill_body, NP2))

    sc_scatter = functools.partial(
        pl.kernel,
        out_type=(),
        mesh=mesh,
        scratch_types=[
            pltpu.VMEM((nch, ch), jnp.int32),
            pltpu.VMEM((ch, DD), jnp.float32),
            pltpu.VMEM((ch, DD), jnp.float32),
            pltpu.SemaphoreType.DMA,
            pltpu.SemaphoreType.DMA,
            pltpu.SemaphoreType.DMA,
            pltpu.SemaphoreType.DMA,
        ],
    )(functools.partial(_sc_scatter_body, rpw, ch))

    out_fill = sc_fill(base)
    out_ref = jax.new_ref(out_fill)
    sc_scatter(v.reshape(rows, DD), gid.reshape(nw, nch, ch), out_ref)
    return jax.freeze(out_ref).reshape(B, NP2, DD)
